# Initial kernel scaffold; baseline (speedup 1.0000x reference)
import jax
import jax.numpy as jnp
from jax.experimental import pallas as pl


def kernel(x, edge_index, w_self_0, w_neigh_0, b_0, w_self_1, w_neigh_1, b_1, w_self_2, w_neigh_2, b_2, w_self_3, w_neigh_3, b_3, w_self_4, w_neigh_4, b_4, w_self_5, w_neigh_5, b_5):
    raise NotImplementedError("write your pallas kernel here")


# speedup vs baseline: 5.9711x; 5.9711x over previous
import functools

import jax
import jax.numpy as jnp
from jax import lax
from jax.experimental import pallas as pl
from jax.experimental.pallas import tpu as pltpu
from jax.experimental.pallas import tpu_sc as plsc

N = 10000
E = 320000
NC = 2
NS = 16
NW = NC * NS
CB = 128
KC = 79
EP = NW * KC * CB
KC2 = NW * KC // NS
N_PAD = 10240
RPT = N_PAD // NS
ZR = 128


def _zero_vmem(buf, rows, width):
    def row(i, c):
        for j in range(width // 16):
            buf[i, pl.ds(j * 16, 16)] = jnp.zeros((16,), jnp.float32)
        return c
    lax.fori_loop(0, rows, row, 0)


def _agg_body(w2, p_hbm, src_hbm, dst_hbm, out_hbm, src_v, dst_v, gbuf, zbuf,
              shared, sem):
    cid = lax.axis_index("c")
    sid = lax.axis_index("s")

    pltpu.sync_copy(src_hbm.at[sid], src_v)
    pltpu.sync_copy(dst_hbm.at[sid], dst_v)

    off = jnp.broadcast_to((cid * N).astype(jnp.int32), (16,))

    def adj(i, c):
        for j in range(CB // 16):
            sl = pl.ds(j * 16, 16)
            src_v[i, sl] = src_v[i, sl] + off
        return c
    lax.fori_loop(0, KC2, adj, 0)

    _zero_vmem(zbuf, ZR, w2)
    for t in range(RPT // ZR):
        pltpu.sync_copy(zbuf, shared.at[pl.ds(sid * RPT + t * ZR, ZR)])
    plsc.subcore_barrier()

    def chunk(j, c):
        pltpu.async_copy(p_hbm.at[src_v.at[j]], gbuf, sem).wait()
        pltpu.sync_copy(gbuf, shared.at[dst_v.at[j]], add=True)
        return c
    lax.fori_loop(0, KC2, chunk, 0)

    plsc.subcore_barrier()
    pltpu.sync_copy(shared.at[pl.ds(sid * RPT, RPT)],
                    out_hbm.at[cid, pl.ds(sid * RPT, RPT)])


@functools.cache
def _make_agg(w2):
    mesh = plsc.VectorSubcoreMesh(core_axis_name="c", subcore_axis_name="s")
    return pl.kernel(
        functools.partial(_agg_body, w2),
        out_type=jax.ShapeDtypeStruct((NC, N_PAD, w2), jnp.float32),
        mesh=mesh,
        compiler_params=pltpu.CompilerParams(use_tc_tiling_on_sc=False),
        scratch_types=[
            pltpu.VMEM((KC2, CB), jnp.int32),
            pltpu.VMEM((KC2, CB), jnp.int32),
            pltpu.VMEM((CB, w2), jnp.float32),
            pltpu.VMEM((ZR, w2), jnp.float32),
            pltpu.VMEM_SHARED((N_PAD, w2), jnp.float32),
            pltpu.SemaphoreType.DMA,
        ],
    )


def _deg_body(dst_hbm, out_hbm, dst_v, ones_v, zbuf, shared):
    cid = lax.axis_index("c")
    sid = lax.axis_index("s")
    wid = cid * NS + sid

    pltpu.sync_copy(dst_hbm.at[wid], dst_v)

    def orow(i, c):
        ones_v[i, pl.ds(0, 16)] = jnp.ones((16,), jnp.float32)
        return c
    lax.fori_loop(0, CB, orow, 0)
    _zero_vmem(zbuf, ZR, 16)
    for t in range(RPT // ZR):
        pltpu.sync_copy(zbuf, shared.at[pl.ds(sid * RPT + t * ZR, ZR)])
    plsc.subcore_barrier()

    def chunk(j, c):
        pltpu.sync_copy(ones_v, shared.at[dst_v.at[j]], add=True)
        return c
    lax.fori_loop(0, KC, chunk, 0)

    plsc.subcore_barrier()
    pltpu.sync_copy(shared.at[pl.ds(sid * RPT, RPT)],
                    out_hbm.at[cid, pl.ds(sid * RPT, RPT)])


@functools.cache
def _make_deg():
    mesh = plsc.VectorSubcoreMesh(core_axis_name="c", subcore_axis_name="s")
    return pl.kernel(
        _deg_body,
        out_type=jax.ShapeDtypeStruct((NC, N_PAD, 16), jnp.float32),
        mesh=mesh,
        compiler_params=pltpu.CompilerParams(use_tc_tiling_on_sc=False),
        scratch_types=[
            pltpu.VMEM((KC, CB), jnp.int32),
            pltpu.VMEM((CB, 16), jnp.float32),
            pltpu.VMEM((ZR, 16), jnp.float32),
            pltpu.VMEM_SHARED((N_PAD, 16), jnp.float32),
        ],
    )


BN = 2000


def _proj_body(h_ref, w_ref, o_ref):
    p = jnp.dot(h_ref[...], w_ref[...], preferred_element_type=jnp.float32,
                precision=lax.Precision.HIGHEST)
    w2 = p.shape[1] // 2
    o_ref[0] = p[:, :w2]
    o_ref[1] = p[:, w2:]


def _project(h, w):
    n, din = h.shape
    dout = w.shape[1]
    w2 = dout // 2
    return pl.pallas_call(
        _proj_body,
        grid=(n // BN,),
        in_specs=[pl.BlockSpec((BN, din), lambda i: (i, 0)),
                  pl.BlockSpec((din, dout), lambda i: (0, 0))],
        out_specs=pl.BlockSpec((NC, BN, w2), lambda i: (0, i, 0)),
        out_shape=jax.ShapeDtypeStruct((NC, n, w2), jnp.float32),
    )(h, w)


def _combine_body(relu, concat, h_ref, ws_ref, b_ref, agg_ref, deg_ref, o_ref):
    h = h_ref[...]
    s = jnp.dot(h, ws_ref[...], preferred_element_type=jnp.float32,
                precision=lax.Precision.HIGHEST) + b_ref[...]
    a = jnp.concatenate([agg_ref[0], agg_ref[1]], axis=1)
    dg = deg_ref[0] + deg_ref[1]
    inv = 1.0 / jnp.maximum(dg[:, 0:1], 1.0)
    r = s + a * inv
    if concat:
        r = jnp.concatenate([r, h], axis=1)
    if relu:
        r = jnp.maximum(r, 0.0)
    o_ref[...] = r


def _combine(h, ws, b, agg, deg, relu, concat):
    n, din = h.shape
    dout = ws.shape[1]
    w2 = dout // 2
    dres = dout + (din if concat else 0)
    return pl.pallas_call(
        functools.partial(_combine_body, relu, concat),
        grid=(n // BN,),
        in_specs=[
            pl.BlockSpec((BN, din), lambda i: (i, 0)),
            pl.BlockSpec((din, dout), lambda i: (0, 0)),
            pl.BlockSpec((1, dout), lambda i: (0, 0)),
            pl.BlockSpec((NC, BN, w2), lambda i: (0, i, 0)),
            pl.BlockSpec((NC, BN, 16), lambda i: (0, i, 0)),
        ],
        out_specs=pl.BlockSpec((BN, dres), lambda i: (i, 0)),
        out_shape=jax.ShapeDtypeStruct((n, dres), jnp.float32),
    )(h, ws, b, agg, deg)


def kernel(x, edge_index, w_self_0, w_neigh_0, b_0, w_self_1, w_neigh_1, b_1,
           w_self_2, w_neigh_2, b_2, w_self_3, w_neigh_3, b_3,
           w_self_4, w_neigh_4, b_4, w_self_5, w_neigh_5, b_5):
    src = edge_index[0]
    dst = edge_index[1]
    pad = EP - E
    srcp = jnp.concatenate([src, jnp.zeros((pad,), jnp.int32)])
    dstp = jnp.concatenate([dst, jnp.full((pad,), N, jnp.int32)])
    src2 = srcp.reshape(NS, KC2, CB)
    dst2 = dstp.reshape(NS, KC2, CB)
    dst3 = dstp.reshape(NW, KC, CB)

    deg = _make_deg()(dst3)

    layers = [
        (w_self_0, w_neigh_0, b_0, True, False),
        (w_self_1, w_neigh_1, b_1, True, False),
        (w_self_2, w_neigh_2, b_2, True, False),
        (w_self_3, w_neigh_3, b_3, True, True),
        (w_self_4, w_neigh_4, b_4, True, True),
        (w_self_5, w_neigh_5, b_5, False, False),
    ]

    h = x
    for ws, wn, b, relu, concat in layers:
        p = _project(h, wn)
        w2 = p.shape[2]
        agg = _make_agg(w2)(p.reshape(NC * N, w2), src2, dst2)
        h = _combine(h, ws, b.reshape(1, -1), agg, deg, relu, concat)
    return h


# speedup vs baseline: 6.1937x; 1.0373x over previous
import functools

import jax
import jax.numpy as jnp
from jax import lax
from jax.experimental import pallas as pl
from jax.experimental.pallas import tpu as pltpu
from jax.experimental.pallas import tpu_sc as plsc

N = 10000
E = 320000
NC = 2
NS = 16
NW = NC * NS
CB = 128
KC2 = 160
EP = NS * KC2 * CB
KC = EP // (NW * CB)
PIPE = 4
NGRP = KC2 // PIPE
N_PAD = 10240
RPT = N_PAD // NS
ZR = 128


def _zero_vmem(buf, rows, width):
    def row(i, c):
        for j in range(width // 16):
            buf[i, pl.ds(j * 16, 16)] = jnp.zeros((16,), jnp.float32)
        return c
    lax.fori_loop(0, rows, row, 0)


def _agg_body(w2, p_hbm, src_hbm, dst_hbm, out_hbm, src_v, dst_v, gbuf, zbuf,
              shared, gsem0, gsem1, gsem2, gsem3, ssem):
    cid = lax.axis_index("c")
    sid = lax.axis_index("s")

    pltpu.sync_copy(src_hbm.at[sid], src_v)
    pltpu.sync_copy(dst_hbm.at[sid], dst_v)

    off = jnp.broadcast_to((cid * N).astype(jnp.int32), (16,))

    def adj(i, c):
        for j in range(CB // 16):
            sl = pl.ds(j * 16, 16)
            src_v[i, sl] = src_v[i, sl] + off
        return c
    lax.fori_loop(0, KC2, adj, 0)

    _zero_vmem(zbuf, ZR, w2)
    for t in range(RPT // ZR):
        pltpu.sync_copy(zbuf, shared.at[pl.ds(sid * RPT + t * ZR, ZR)])
    plsc.subcore_barrier()

    gsems = (gsem0, gsem1, gsem2, gsem3)

    def group(g, c):
        gds = [pltpu.async_copy(p_hbm.at[src_v.at[g * PIPE + b]],
                                gbuf.at[pl.ds(b * CB, CB)], gsems[b])
               for b in range(PIPE)]
        sds = []
        for b in range(PIPE):
            gds[b].wait()
            sds.append(pltpu.async_copy(gbuf.at[pl.ds(b * CB, CB)],
                                        shared.at[dst_v.at[g * PIPE + b]],
                                        ssem, add=True))
        for d in sds:
            d.wait()
        return c
    lax.fori_loop(0, NGRP, group, 0)

    plsc.subcore_barrier()
    pltpu.sync_copy(shared.at[pl.ds(sid * RPT, RPT)],
                    out_hbm.at[cid, pl.ds(sid * RPT, RPT)])


@functools.cache
def _make_agg(w2):
    mesh = plsc.VectorSubcoreMesh(core_axis_name="c", subcore_axis_name="s")
    return pl.kernel(
        functools.partial(_agg_body, w2),
        out_type=jax.ShapeDtypeStruct((NC, N_PAD, w2), jnp.float32),
        mesh=mesh,
        compiler_params=pltpu.CompilerParams(use_tc_tiling_on_sc=False,
                                            has_side_effects=True),
        scratch_types=[
            pltpu.VMEM((KC2, CB), jnp.int32),
            pltpu.VMEM((KC2, CB), jnp.int32),
            pltpu.VMEM((PIPE * CB, w2), jnp.float32),
            pltpu.VMEM((ZR, w2), jnp.float32),
            pltpu.VMEM_SHARED((N_PAD, w2), jnp.float32),
            pltpu.SemaphoreType.DMA,
            pltpu.SemaphoreType.DMA,
            pltpu.SemaphoreType.DMA,
            pltpu.SemaphoreType.DMA,
            pltpu.SemaphoreType.DMA,
        ],
    )


def _deg_body(dst_hbm, out_hbm, dst_v, ones_v, zbuf, shared, sem):
    cid = lax.axis_index("c")
    sid = lax.axis_index("s")
    wid = cid * NS + sid

    pltpu.sync_copy(dst_hbm.at[wid], dst_v)

    def orow(i, c):
        ones_v[i, pl.ds(0, 16)] = jnp.ones((16,), jnp.float32)
        return c
    lax.fori_loop(0, CB, orow, 0)
    _zero_vmem(zbuf, ZR, 16)
    for t in range(RPT // ZR):
        pltpu.sync_copy(zbuf, shared.at[pl.ds(sid * RPT + t * ZR, ZR)])
    plsc.subcore_barrier()

    def chunk(j, c):
        pltpu.sync_copy(ones_v, shared.at[dst_v.at[j]], add=True)
        return c
    lax.fori_loop(0, KC, chunk, 0)

    plsc.subcore_barrier()
    pltpu.sync_copy(shared.at[pl.ds(sid * RPT, RPT)],
                    out_hbm.at[cid, pl.ds(sid * RPT, RPT)])


@functools.cache
def _make_deg():
    mesh = plsc.VectorSubcoreMesh(core_axis_name="c", subcore_axis_name="s")
    return pl.kernel(
        _deg_body,
        out_type=jax.ShapeDtypeStruct((NC, N_PAD, 16), jnp.float32),
        mesh=mesh,
        compiler_params=pltpu.CompilerParams(use_tc_tiling_on_sc=False,
                                            has_side_effects=True),
        scratch_types=[
            pltpu.VMEM((KC, CB), jnp.int32),
            pltpu.VMEM((CB, 16), jnp.float32),
            pltpu.VMEM((ZR, 16), jnp.float32),
            pltpu.VMEM_SHARED((N_PAD, 16), jnp.float32),
            pltpu.SemaphoreType.DMA,
        ],
    )


BN = 2000


def _proj_body(h_ref, w_ref, o_ref):
    p = jnp.dot(h_ref[...], w_ref[...], preferred_element_type=jnp.float32,
                precision=lax.Precision.HIGHEST)
    w2 = p.shape[1] // 2
    o_ref[0] = p[:, :w2]
    o_ref[1] = p[:, w2:]


def _project(h, w):
    n, din = h.shape
    dout = w.shape[1]
    w2 = dout // 2
    return pl.pallas_call(
        _proj_body,
        grid=(n // BN,),
        in_specs=[pl.BlockSpec((BN, din), lambda i: (i, 0)),
                  pl.BlockSpec((din, dout), lambda i: (0, 0))],
        out_specs=pl.BlockSpec((NC, BN, w2), lambda i: (0, i, 0)),
        out_shape=jax.ShapeDtypeStruct((NC, n, w2), jnp.float32),
    )(h, w)


def _combine_body(relu, concat, h_ref, ws_ref, b_ref, agg_ref, deg_ref, o_ref):
    h = h_ref[...]
    s = jnp.dot(h, ws_ref[...], preferred_element_type=jnp.float32,
                precision=lax.Precision.HIGHEST) + b_ref[...]
    a = jnp.concatenate([agg_ref[0], agg_ref[1]], axis=1)
    dg = deg_ref[0] + deg_ref[1]
    inv = 1.0 / jnp.maximum(dg[:, 0:1], 1.0)
    r = s + a * inv
    if concat:
        r = jnp.concatenate([r, h], axis=1)
    if relu:
        r = jnp.maximum(r, 0.0)
    o_ref[...] = r


def _combine(h, ws, b, agg, deg, relu, concat):
    n, din = h.shape
    dout = ws.shape[1]
    w2 = dout // 2
    dres = dout + (din if concat else 0)
    return pl.pallas_call(
        functools.partial(_combine_body, relu, concat),
        grid=(n // BN,),
        in_specs=[
            pl.BlockSpec((BN, din), lambda i: (i, 0)),
            pl.BlockSpec((din, dout), lambda i: (0, 0)),
            pl.BlockSpec((1, dout), lambda i: (0, 0)),
            pl.BlockSpec((NC, BN, w2), lambda i: (0, i, 0)),
            pl.BlockSpec((NC, BN, 16), lambda i: (0, i, 0)),
        ],
        out_specs=pl.BlockSpec((BN, dres), lambda i: (i, 0)),
        out_shape=jax.ShapeDtypeStruct((n, dres), jnp.float32),
    )(h, ws, b, agg, deg)


def kernel(x, edge_index, w_self_0, w_neigh_0, b_0, w_self_1, w_neigh_1, b_1,
           w_self_2, w_neigh_2, b_2, w_self_3, w_neigh_3, b_3,
           w_self_4, w_neigh_4, b_4, w_self_5, w_neigh_5, b_5):
    src = edge_index[0]
    dst = edge_index[1]
    pad = EP - E
    srcp = jnp.concatenate([src, jnp.zeros((pad,), jnp.int32)])
    dstp = jnp.concatenate([dst, jnp.full((pad,), N, jnp.int32)])
    src2 = srcp.reshape(NS, KC2, CB)
    dst2 = dstp.reshape(NS, KC2, CB)
    dst3 = dstp.reshape(NW, KC, CB)

    deg = _make_deg()(dst3)

    layers = [
        (w_self_0, w_neigh_0, b_0, True, False),
        (w_self_1, w_neigh_1, b_1, True, False),
        (w_self_2, w_neigh_2, b_2, True, False),
        (w_self_3, w_neigh_3, b_3, True, True),
        (w_self_4, w_neigh_4, b_4, True, True),
        (w_self_5, w_neigh_5, b_5, False, False),
    ]

    h = x
    for ws, wn, b, relu, concat in layers:
        p = _project(h, wn)
        w2 = p.shape[2]
        agg = _make_agg(w2)(p.reshape(NC * N, w2), src2, dst2)
        h = _combine(h, ws, b.reshape(1, -1), agg, deg, relu, concat)
    return h


# speedup vs baseline: 6.5420x; 1.0562x over previous
import functools

import jax
import jax.numpy as jnp
from jax import lax
from jax.experimental import pallas as pl
from jax.experimental.pallas import tpu as pltpu
from jax.experimental.pallas import tpu_sc as plsc

N = 10000
E = 320000
NC = 2
NS = 16
NW = NC * NS
CB = 128
KC2 = 160
EP = NS * KC2 * CB
KC = EP // (NW * CB)
PIPE = 4
NGRP = KC2 // PIPE
N_PAD = 10240
RPT = N_PAD // NS
ZR = 128


def _zero_vmem(buf, rows, width):
    def row(i, c):
        for j in range(width // 16):
            buf[i, pl.ds(j * 16, 16)] = jnp.zeros((16,), jnp.float32)
        return c
    lax.fori_loop(0, rows, row, 0)


def _agg_body(w2, p_hbm, src_hbm, dst_hbm, out_hbm, src_v, dst_v, gbuf,
              zbuf, shared, gsem0, gsem1, gsem2, gsem3, ssem):
    cid = lax.axis_index("c")
    sid = lax.axis_index("s")

    pltpu.sync_copy(src_hbm.at[sid], src_v)
    pltpu.sync_copy(dst_hbm.at[sid], dst_v)

    off = jnp.broadcast_to((cid * N).astype(jnp.int32), (16,))

    def adj(i, c):
        for j in range(CB // 16):
            sl = pl.ds(j * 16, 16)
            src_v[i, sl] = src_v[i, sl] + off
        return c
    lax.fori_loop(0, KC2, adj, 0)

    _zero_vmem(zbuf, ZR, w2)
    for t in range(RPT // ZR):
        pltpu.sync_copy(zbuf, shared.at[pl.ds(sid * RPT + t * ZR, ZR)])
    plsc.subcore_barrier()

    gsems = (gsem0, gsem1, gsem2, gsem3)

    def group(g, c):
        gds = [pltpu.async_copy(p_hbm.at[src_v.at[g * PIPE + b]],
                                gbuf.at[pl.ds(b * CB, CB)], gsems[b])
               for b in range(PIPE)]
        sds = []
        for b in range(PIPE):
            gds[b].wait()
            sds.append(pltpu.async_copy(gbuf.at[pl.ds(b * CB, CB)],
                                        shared.at[dst_v.at[g * PIPE + b]],
                                        ssem, add=True))
        for d in sds:
            d.wait()
        return c
    lax.fori_loop(0, NGRP, group, 0)

    plsc.subcore_barrier()
    pltpu.sync_copy(shared.at[pl.ds(sid * RPT, RPT)],
                    out_hbm.at[cid, pl.ds(sid * RPT, RPT)])


@functools.cache
def _make_agg(w2):
    mesh = plsc.VectorSubcoreMesh(core_axis_name="c", subcore_axis_name="s")
    return pl.kernel(
        functools.partial(_agg_body, w2),
        out_type=jax.ShapeDtypeStruct((NC, N_PAD, w2), jnp.float32),
        mesh=mesh,
        compiler_params=pltpu.CompilerParams(use_tc_tiling_on_sc=False,
                                            has_side_effects=True),
        scratch_types=[
            pltpu.VMEM((KC2, CB), jnp.int32),
            pltpu.VMEM((KC2, CB), jnp.int32),
            pltpu.VMEM((PIPE * CB, w2), jnp.float32),
            pltpu.VMEM((ZR, w2), jnp.float32),
            pltpu.VMEM_SHARED((N_PAD, w2), jnp.float32),
            pltpu.SemaphoreType.DMA,
            pltpu.SemaphoreType.DMA,
            pltpu.SemaphoreType.DMA,
            pltpu.SemaphoreType.DMA,
            pltpu.SemaphoreType.DMA,
        ],
    )


def _deg_body(dst_hbm, out_hbm, dst_v, ones_v, zbuf, shared):
    cid = lax.axis_index("c")
    sid = lax.axis_index("s")
    wid = cid * NS + sid

    pltpu.sync_copy(dst_hbm.at[wid], dst_v)

    def orow(i, c):
        ones_v[i, pl.ds(0, 16)] = jnp.ones((16,), jnp.float32)
        return c
    lax.fori_loop(0, CB, orow, 0)
    _zero_vmem(zbuf, ZR, 16)
    for t in range(RPT // ZR):
        pltpu.sync_copy(zbuf, shared.at[pl.ds(sid * RPT + t * ZR, ZR)])
    plsc.subcore_barrier()

    def chunk(j, c):
        pltpu.sync_copy(ones_v, shared.at[dst_v.at[j]], add=True)
        return c
    lax.fori_loop(0, KC, chunk, 0)

    plsc.subcore_barrier()
    pltpu.sync_copy(shared.at[pl.ds(sid * RPT, RPT)],
                    out_hbm.at[cid, pl.ds(sid * RPT, RPT)])


@functools.cache
def _make_deg():
    mesh = plsc.VectorSubcoreMesh(core_axis_name="c", subcore_axis_name="s")
    return pl.kernel(
        _deg_body,
        out_type=jax.ShapeDtypeStruct((NC, N_PAD, 16), jnp.float32),
        mesh=mesh,
        compiler_params=pltpu.CompilerParams(use_tc_tiling_on_sc=False,
                                            has_side_effects=True),
        scratch_types=[
            pltpu.VMEM((KC, CB), jnp.int32),
            pltpu.VMEM((CB, 16), jnp.float32),
            pltpu.VMEM((ZR, 16), jnp.float32),
            pltpu.VMEM_SHARED((N_PAD, 16), jnp.float32),
        ],
    )


BN = 2000


def _proj_body(h_ref, w_ref, o_ref):
    p = jnp.dot(h_ref[...], w_ref[...], preferred_element_type=jnp.float32,
                precision=lax.Precision.HIGHEST)
    w2 = p.shape[1] // 2
    o_ref[0] = p[:, :w2]
    o_ref[1] = p[:, w2:]


def _project(h, w):
    n, din = h.shape
    dout = w.shape[1]
    w2 = dout // 2
    return pl.pallas_call(
        _proj_body,
        grid=(n // BN,),
        in_specs=[pl.BlockSpec((BN, din), lambda i: (i, 0)),
                  pl.BlockSpec((din, dout), lambda i: (0, 0))],
        out_specs=pl.BlockSpec((NC, BN, w2), lambda i: (0, i, 0)),
        out_shape=jax.ShapeDtypeStruct((NC, n, w2), jnp.float32),
    )(h, w)


def _combine_body(relu, concat, h_ref, ws_ref, b_ref, agg_ref, deg_ref, o_ref):
    h = h_ref[...]
    s = jnp.dot(h, ws_ref[...], preferred_element_type=jnp.float32,
                precision=lax.Precision.HIGHEST) + b_ref[...]
    a = jnp.concatenate([agg_ref[0], agg_ref[1]], axis=1)
    dg = deg_ref[0] + deg_ref[1]
    inv = 1.0 / jnp.maximum(dg[:, 0:1], 1.0)
    r = s + a * inv
    if concat:
        r = jnp.concatenate([r, h], axis=1)
    if relu:
        r = jnp.maximum(r, 0.0)
    o_ref[...] = r


def _combine_proj_body(relu, concat, h_ref, ws_ref, b_ref, agg_ref, deg_ref,
                       wn_ref, o_ref, p_ref):
    h = h_ref[...]
    s = jnp.dot(h, ws_ref[...], preferred_element_type=jnp.float32,
                precision=lax.Precision.HIGHEST) + b_ref[...]
    a = jnp.concatenate([agg_ref[0], agg_ref[1]], axis=1)
    dg = deg_ref[0] + deg_ref[1]
    inv = 1.0 / jnp.maximum(dg[:, 0:1], 1.0)
    r = s + a * inv
    if concat:
        r = jnp.concatenate([r, h], axis=1)
    if relu:
        r = jnp.maximum(r, 0.0)
    o_ref[...] = r
    p = jnp.dot(r, wn_ref[...], preferred_element_type=jnp.float32,
                precision=lax.Precision.HIGHEST)
    w2n = p.shape[1] // 2
    p_ref[0] = p[:, :w2n]
    p_ref[1] = p[:, w2n:]


def _combine_proj(h, ws, b, agg, deg, wn_next, relu, concat):
    n, din = h.shape
    dout = ws.shape[1]
    w2 = dout // 2
    dres = dout + (din if concat else 0)
    w2n = wn_next.shape[1] // 2
    return pl.pallas_call(
        functools.partial(_combine_proj_body, relu, concat),
        grid=(n // BN,),
        in_specs=[
            pl.BlockSpec((BN, din), lambda i: (i, 0)),
            pl.BlockSpec((din, dout), lambda i: (0, 0)),
            pl.BlockSpec((1, dout), lambda i: (0, 0)),
            pl.BlockSpec((NC, BN, w2), lambda i: (0, i, 0)),
            pl.BlockSpec((NC, BN, 16), lambda i: (0, i, 0)),
            pl.BlockSpec((dres, 2 * w2n), lambda i: (0, 0)),
        ],
        out_specs=[pl.BlockSpec((BN, dres), lambda i: (i, 0)),
                   pl.BlockSpec((NC, BN, w2n), lambda i: (0, i, 0))],
        out_shape=[jax.ShapeDtypeStruct((n, dres), jnp.float32),
                   jax.ShapeDtypeStruct((NC, n, w2n), jnp.float32)],
    )(h, ws, b, agg, deg, wn_next)


def _combine(h, ws, b, agg, deg, relu, concat):
    n, din = h.shape
    dout = ws.shape[1]
    w2 = dout // 2
    dres = dout + (din if concat else 0)
    return pl.pallas_call(
        functools.partial(_combine_body, relu, concat),
        grid=(n // BN,),
        in_specs=[
            pl.BlockSpec((BN, din), lambda i: (i, 0)),
            pl.BlockSpec((din, dout), lambda i: (0, 0)),
            pl.BlockSpec((1, dout), lambda i: (0, 0)),
            pl.BlockSpec((NC, BN, w2), lambda i: (0, i, 0)),
            pl.BlockSpec((NC, BN, 16), lambda i: (0, i, 0)),
        ],
        out_specs=pl.BlockSpec((BN, dres), lambda i: (i, 0)),
        out_shape=jax.ShapeDtypeStruct((n, dres), jnp.float32),
    )(h, ws, b, agg, deg)


def kernel(x, edge_index, w_self_0, w_neigh_0, b_0, w_self_1, w_neigh_1, b_1,
           w_self_2, w_neigh_2, b_2, w_self_3, w_neigh_3, b_3,
           w_self_4, w_neigh_4, b_4, w_self_5, w_neigh_5, b_5):
    src = edge_index[0]
    dst = edge_index[1]
    pad = EP - E
    srcp = jnp.concatenate([src, jnp.zeros((pad,), jnp.int32)])
    dstp = jnp.concatenate([dst, jnp.full((pad,), N, jnp.int32)])
    src2 = srcp.reshape(NS, KC2, CB)
    dst2 = dstp.reshape(NS, KC2, CB)
    dst3 = dstp.reshape(NW, KC, CB)

    deg = _make_deg()(dst3)

    layers = [
        (w_self_0, w_neigh_0, b_0, True, False),
        (w_self_1, w_neigh_1, b_1, True, False),
        (w_self_2, w_neigh_2, b_2, True, False),
        (w_self_3, w_neigh_3, b_3, True, True),
        (w_self_4, w_neigh_4, b_4, True, True),
        (w_self_5, w_neigh_5, b_5, False, False),
    ]

    h = x
    p = _project(x, w_neigh_0)
    for li, (ws, wn, b, relu, concat) in enumerate(layers):
        agg = _make_agg(p.shape[2])(p.reshape(NC * N, p.shape[2]), src2, dst2)
        if li + 1 < len(layers):
            h, p = _combine_proj(h, ws, b.reshape(1, -1), agg, deg,
                                 layers[li + 1][1], relu, concat)
        else:
            h = _combine(h, ws, b.reshape(1, -1), agg, deg, relu, concat)
    return h
